# baseline (speedup 1.0000x reference)
import functools

import jax
import jax.numpy as jnp
from jax import lax
from jax.experimental import pallas as pl
from jax.experimental.pallas import tpu as pltpu
from jax.experimental.pallas import tpu_sc as plsc

EPS_ = 1e-07
LN_EPS_ = 1e-12


def _entity_gather(table, ids):
    V, H = table.shape
    NW, n_ch, CH = ids.shape
    BM = NW * n_ch * CH
    b_per_w = n_ch * CH
    info = plsc.get_sparse_core_info()
    NC = info.num_cores
    mesh = plsc.VectorSubcoreMesh(core_axis_name="c", subcore_axis_name="s")

    @functools.partial(
        pl.kernel,
        mesh=mesh,
        out_type=jax.ShapeDtypeStruct((BM, H), jnp.float32),
        scratch_types=[
            pltpu.VMEM((n_ch, CH), jnp.int32),
            pltpu.VMEM((CH, H), jnp.float32),
            pltpu.SemaphoreType.DMA,
        ],
    )
    def gather_k(table_hbm, idx_hbm, out_hbm, idx_v, buf, sem):
        wid = lax.axis_index("s") * NC + lax.axis_index("c")
        base = wid * b_per_w
        pltpu.sync_copy(idx_hbm.at[wid], idx_v)
        for c in range(n_ch):
            pltpu.async_copy(table_hbm.at[idx_v.at[c]], buf, sem).wait()
            pltpu.sync_copy(buf, out_hbm.at[pl.ds(base + c * CH, CH)])

    return gather_k(table, ids)


def _pos_tt(ids_all, pos_table, tt_table, L):
    BM = ids_all.shape[0]
    P, H = pos_table.shape
    R = 256
    nb = BM // R

    def body(ids_ref, pt_ref, tt_ref, o_ref):
        ids = ids_ref[...]
        cols = lax.broadcasted_iota(jnp.int16, (R, P), 1)
        ids16 = ids.astype(jnp.int16)
        counts = jnp.zeros((R, P), jnp.int16)
        valid = jnp.zeros((R, 1), jnp.float32)
        for l in range(L):
            idl = ids16[:, l:l + 1]
            counts = counts + (idl == cols).astype(jnp.int16)
            valid = valid + (ids[:, l:l + 1] != -1).astype(jnp.float32)
        pos_sum = jnp.dot(counts.astype(jnp.bfloat16), pt_ref[...],
                          preferred_element_type=jnp.float32)
        pooled = pos_sum / jnp.maximum(valid, EPS_)
        ttf = ids[:, L:L + 1].astype(jnp.float32)
        tt0 = tt_ref[0:1, :]
        tt1 = tt_ref[1:2, :]
        o_ref[...] = pooled + tt0 + ttf * (tt1 - tt0)

    return pl.pallas_call(
        body,
        grid=(nb,),
        in_specs=[
            pl.BlockSpec((R, 32), lambda i: (i, 0)),
            pl.BlockSpec((P, H), lambda i: (0, 0)),
            pl.BlockSpec((tt_table.shape[0], H), lambda i: (0, 0)),
        ],
        out_specs=pl.BlockSpec((R, H), lambda i: (i, 0)),
        out_shape=jax.ShapeDtypeStruct((BM, H), jnp.float32),
    )(ids_all, pos_table, tt_table)


def _add_ln(ent_emb, pos_tt, gamma, beta):
    BM, H = ent_emb.shape
    R = 512
    nb = BM // R

    def body(ent_ref, pt_ref, g_ref, b_ref, o_ref):
        emb = ent_ref[...] + pt_ref[...]
        mean = jnp.mean(emb, axis=1, keepdims=True)
        cent = emb - mean
        var = jnp.mean(cent * cent, axis=1, keepdims=True)
        o_ref[...] = (cent * lax.rsqrt(var + LN_EPS_) * g_ref[...]
                      + b_ref[...])

    return pl.pallas_call(
        body,
        grid=(nb,),
        in_specs=[
            pl.BlockSpec((R, H), lambda i: (i, 0)),
            pl.BlockSpec((R, H), lambda i: (i, 0)),
            pl.BlockSpec((1, H), lambda i: (0, 0)),
            pl.BlockSpec((1, H), lambda i: (0, 0)),
        ],
        out_specs=pl.BlockSpec((R, H), lambda i: (i, 0)),
        out_shape=jax.ShapeDtypeStruct((BM, H), jnp.float32),
    )(ent_emb, pos_tt, gamma, beta)


def kernel(entity_ids, position_ids, token_type_ids, entity_table,
           position_table, token_type_table, gamma, beta):
    B, M = entity_ids.shape
    L = position_ids.shape[-1]
    H = entity_table.shape[1]
    BM = B * M

    NW, CH = 32, 128
    eids = entity_ids.reshape(NW, BM // (NW * CH), CH).astype(jnp.int32)
    ent_emb = _entity_gather(entity_table, eids)

    pos = position_ids.reshape(BM, L).astype(jnp.int32)
    tt = token_type_ids.reshape(BM, 1).astype(jnp.int32)
    pad = jnp.full((BM, 32 - L - 1), -1, jnp.int32)
    ids_all = jnp.concatenate([pos, tt, pad], axis=1)

    pos_tt = _pos_tt(ids_all, position_table.astype(jnp.bfloat16),
                     token_type_table, L)
    out = _add_ln(ent_emb, pos_tt, gamma.reshape(1, H), beta.reshape(1, H))
    return out.reshape(B, M, H)


# speedup vs baseline: 1.3369x; 1.3369x over previous
import functools

import jax
import jax.numpy as jnp
from jax import lax
from jax.experimental import pallas as pl
from jax.experimental.pallas import tpu as pltpu
from jax.experimental.pallas import tpu_sc as plsc

EPS_ = 1e-07
LN_EPS_ = 1e-12


def _entity_gather(table, ids):
    V, H = table.shape
    NW, n_ch, CH = ids.shape
    BM = NW * n_ch * CH
    b_per_w = n_ch * CH
    info = plsc.get_sparse_core_info()
    NC = info.num_cores
    mesh = plsc.VectorSubcoreMesh(core_axis_name="c", subcore_axis_name="s")

    @functools.partial(
        pl.kernel,
        mesh=mesh,
        out_type=jax.ShapeDtypeStruct((BM, H), jnp.float32),
        scratch_types=[
            pltpu.VMEM((n_ch, CH), jnp.int32),
            pltpu.VMEM((CH, H), jnp.float32),
            pltpu.SemaphoreType.DMA,
        ],
    )
    def gather_k(table_hbm, idx_hbm, out_hbm, idx_v, buf, sem):
        wid = lax.axis_index("s") * NC + lax.axis_index("c")
        base = wid * b_per_w
        pltpu.sync_copy(idx_hbm.at[wid], idx_v)
        for c in range(n_ch):
            pltpu.async_copy(table_hbm.at[idx_v.at[c]], buf, sem).wait()
            pltpu.sync_copy(buf, out_hbm.at[pl.ds(base + c * CH, CH)])

    return gather_k(table, ids)


def _fuse(ent_emb, ids_all, pos_table, tt_table, gamma, beta, L, B, M):
    BM = ids_all.shape[0]
    P, H = pos_table.shape
    BB = 16
    R = BB * M
    nb = B // BB

    def body(ids_ref, ent_ref, pt_ref, tt_ref, g_ref, b_ref, o_ref):
        ids = ids_ref[...]
        cols = lax.broadcasted_iota(jnp.int16, (R, P), 1)
        ids16 = ids.astype(jnp.int16)
        counts = jnp.zeros((R, P), jnp.int16)
        valid = jnp.zeros((R, 1), jnp.float32)
        for l in range(L):
            idl = ids16[:, l:l + 1]
            counts = counts + (idl == cols).astype(jnp.int16)
            valid = valid + (ids[:, l:l + 1] != -1).astype(jnp.float32)
        pos_sum = jnp.dot(counts.astype(jnp.bfloat16), pt_ref[...],
                          preferred_element_type=jnp.float32)
        pooled = pos_sum / jnp.maximum(valid, EPS_)
        ttf = ids[:, L:L + 1].astype(jnp.float32)
        tt0 = tt_ref[0:1, :]
        tt1 = tt_ref[1:2, :]
        emb = ent_ref[...] + pooled + tt0 + ttf * (tt1 - tt0)
        mean = jnp.mean(emb, axis=1, keepdims=True)
        cent = emb - mean
        var = jnp.mean(cent * cent, axis=1, keepdims=True)
        out = (cent * lax.rsqrt(var + LN_EPS_) * g_ref[...]
               + b_ref[...])
        o_ref[...] = out.reshape(BB, M, H)

    return pl.pallas_call(
        body,
        grid=(nb,),
        in_specs=[
            pl.BlockSpec((R, 32), lambda i: (i, 0)),
            pl.BlockSpec((R, H), lambda i: (i, 0)),
            pl.BlockSpec((P, H), lambda i: (0, 0)),
            pl.BlockSpec((tt_table.shape[0], H), lambda i: (0, 0)),
            pl.BlockSpec((1, H), lambda i: (0, 0)),
            pl.BlockSpec((1, H), lambda i: (0, 0)),
        ],
        out_specs=pl.BlockSpec((BB, M, H), lambda i: (i, 0, 0)),
        out_shape=jax.ShapeDtypeStruct((B, M, H), jnp.float32),
    )(ids_all, ent_emb, pos_table, tt_table, gamma, beta)


def kernel(entity_ids, position_ids, token_type_ids, entity_table,
           position_table, token_type_table, gamma, beta):
    B, M = entity_ids.shape
    L = position_ids.shape[-1]
    H = entity_table.shape[1]
    BM = B * M

    NW, CH = 32, 128
    eids = entity_ids.reshape(NW, BM // (NW * CH), CH).astype(jnp.int32)
    ent_emb = _entity_gather(entity_table, eids)

    pos = position_ids.reshape(BM, L).astype(jnp.int32)
    tt = token_type_ids.reshape(BM, 1).astype(jnp.int32)
    pad = jnp.full((BM, 32 - L - 1), -1, jnp.int32)
    ids_all = jnp.concatenate([pos, tt, pad], axis=1)

    return _fuse(ent_emb, ids_all, position_table.astype(jnp.bfloat16),
                 token_type_table, gamma.reshape(1, H), beta.reshape(1, H),
                 L, B, M)


# speedup vs baseline: 1.7548x; 1.3126x over previous
import functools

import jax
import jax.numpy as jnp
from jax import lax
from jax.experimental import pallas as pl
from jax.experimental.pallas import tpu as pltpu
from jax.experimental.pallas import tpu_sc as plsc

EPS_ = 1e-07
LN_EPS_ = 1e-12


def _entity_gather(table, ids):
    V, H = table.shape
    NW, n_ch, CH = ids.shape
    BM = NW * n_ch * CH
    b_per_w = n_ch * CH
    info = plsc.get_sparse_core_info()
    NC = info.num_cores
    mesh = plsc.VectorSubcoreMesh(core_axis_name="c", subcore_axis_name="s")

    @functools.partial(
        pl.kernel,
        mesh=mesh,
        out_type=jax.ShapeDtypeStruct((BM, H), jnp.float32),
        scratch_types=[
            pltpu.VMEM((n_ch, CH), jnp.int32),
            pltpu.VMEM((CH, H), jnp.float32),
            pltpu.SemaphoreType.DMA,
        ],
    )
    def gather_k(table_hbm, idx_hbm, out_hbm, idx_v, buf, sem):
        wid = lax.axis_index("s") * NC + lax.axis_index("c")
        base = wid * b_per_w
        pltpu.sync_copy(idx_hbm.at[wid], idx_v)
        for c in range(n_ch):
            pltpu.async_copy(table_hbm.at[idx_v.at[c]], buf, sem).wait()
            pltpu.sync_copy(buf, out_hbm.at[pl.ds(base + c * CH, CH)])

    return gather_k(table, ids)


def _fuse(ent_emb, ids_all, pos_table, tt_table, gamma, beta, L):
    BM = ids_all.shape[0]
    P, H = pos_table.shape
    R = 512
    nb = BM // R

    def body(ids_ref, ent_ref, pt_ref, tt_ref, g_ref, b_ref, o_ref):
        ids = ids_ref[...]
        cols = lax.broadcasted_iota(jnp.int16, (R, P), 1)
        ids16 = ids.astype(jnp.int16)
        counts = jnp.zeros((R, P), jnp.int16)
        valid = jnp.zeros((R, 1), jnp.float32)
        for l in range(L):
            idl = ids16[:, l:l + 1]
            counts = counts + (idl == cols).astype(jnp.int16)
            valid = valid + (ids[:, l:l + 1] != -1).astype(jnp.float32)
        pos_sum = jnp.dot(counts.astype(jnp.bfloat16), pt_ref[...],
                          preferred_element_type=jnp.float32)
        pooled = pos_sum / jnp.maximum(valid, EPS_)
        ttf = ids[:, L:L + 1].astype(jnp.float32)
        tt0 = tt_ref[0:1, :]
        tt1 = tt_ref[1:2, :]
        emb = ent_ref[...] + pooled + tt0 + ttf * (tt1 - tt0)
        mean = jnp.mean(emb, axis=1, keepdims=True)
        cent = emb - mean
        var = jnp.mean(cent * cent, axis=1, keepdims=True)
        o_ref[...] = (cent * lax.rsqrt(var + LN_EPS_) * g_ref[...]
                      + b_ref[...])

    return pl.pallas_call(
        body,
        grid=(nb,),
        in_specs=[
            pl.BlockSpec((R, 32), lambda i: (i, 0)),
            pl.BlockSpec((R, H), lambda i: (i, 0)),
            pl.BlockSpec((P, H), lambda i: (0, 0)),
            pl.BlockSpec((tt_table.shape[0], H), lambda i: (0, 0)),
            pl.BlockSpec((1, H), lambda i: (0, 0)),
            pl.BlockSpec((1, H), lambda i: (0, 0)),
        ],
        out_specs=pl.BlockSpec((R, H), lambda i: (i, 0)),
        out_shape=jax.ShapeDtypeStruct((BM, H), jnp.float32),
    )(ids_all, ent_emb, pos_table, tt_table, gamma, beta)


def kernel(entity_ids, position_ids, token_type_ids, entity_table,
           position_table, token_type_table, gamma, beta):
    B, M = entity_ids.shape
    L = position_ids.shape[-1]
    H = entity_table.shape[1]
    BM = B * M

    NW, CH = 32, 128
    eids = entity_ids.T.reshape(NW, BM // (NW * CH), CH).astype(jnp.int32)
    ent_emb = _entity_gather(entity_table, eids)

    pos = position_ids.transpose(1, 0, 2).reshape(BM, L).astype(jnp.int32)
    tt = token_type_ids.T.reshape(BM, 1).astype(jnp.int32)
    pad = jnp.full((BM, 32 - L - 1), -1, jnp.int32)
    ids_all = jnp.concatenate([pos, tt, pad], axis=1)

    out = _fuse(ent_emb, ids_all, position_table.astype(jnp.bfloat16),
                token_type_table, gamma.reshape(1, H), beta.reshape(1, H), L)
    return out.reshape(M, B, H).transpose(1, 0, 2)


# speedup vs baseline: 1.7873x; 1.0185x over previous
import functools

import jax
import jax.numpy as jnp
from jax import lax
from jax.experimental import pallas as pl
from jax.experimental.pallas import tpu as pltpu
from jax.experimental.pallas import tpu_sc as plsc

EPS_ = 1e-07
LN_EPS_ = 1e-12


def _entity_gather(table, ids):
    V, H = table.shape
    NW, n_ch, CH = ids.shape
    BM = NW * n_ch * CH
    b_per_w = n_ch * CH
    info = plsc.get_sparse_core_info()
    NC = info.num_cores
    mesh = plsc.VectorSubcoreMesh(core_axis_name="c", subcore_axis_name="s")

    @functools.partial(
        pl.kernel,
        mesh=mesh,
        out_type=jax.ShapeDtypeStruct((BM, H), jnp.float32),
        scratch_types=[
            pltpu.VMEM((n_ch, CH), jnp.int32),
            pltpu.VMEM((CH, H), jnp.float32),
            pltpu.VMEM((CH, H), jnp.float32),
            pltpu.SemaphoreType.DMA,
            pltpu.SemaphoreType.DMA,
            pltpu.SemaphoreType.DMA,
            pltpu.SemaphoreType.DMA,
        ],
    )
    def gather_k(table_hbm, idx_hbm, out_hbm, idx_v, buf0, buf1,
                 gs0, gs1, ss0, ss1):
        wid = lax.axis_index("s") * NC + lax.axis_index("c")
        base = wid * b_per_w
        pltpu.sync_copy(idx_hbm.at[wid], idx_v)
        bufs = (buf0, buf1)
        gsems = (gs0, gs1)
        ssems = (ss0, ss1)
        gcp = [None] * n_ch
        scp = [None] * n_ch
        gcp[0] = pltpu.async_copy(
            table_hbm.at[idx_v.at[0]], bufs[0], gsems[0])
        for c in range(n_ch):
            gcp[c].wait()
            scp[c] = pltpu.async_copy(
                bufs[c % 2], out_hbm.at[pl.ds(base + c * CH, CH)],
                ssems[c % 2])
            if c + 1 < n_ch:
                if c >= 1:
                    scp[c - 1].wait()
                gcp[c + 1] = pltpu.async_copy(
                    table_hbm.at[idx_v.at[c + 1]],
                    bufs[(c + 1) % 2], gsems[(c + 1) % 2])
        scp[n_ch - 1].wait()

    return gather_k(table, ids)


def _fuse(ent_emb, ids_all, pos_table, tt_table, gamma, beta, L):
    BM = ids_all.shape[0]
    P, H = pos_table.shape
    R = 512
    nb = BM // R

    def body(ids_ref, ent_ref, pt_ref, tt_ref, g_ref, b_ref, o_ref):
        ids = ids_ref[...]
        cols = lax.broadcasted_iota(jnp.int16, (R, P), 1)
        ids16 = ids.astype(jnp.int16)
        counts = jnp.zeros((R, P), jnp.int16)
        valid = jnp.zeros((R, 1), jnp.float32)
        for l in range(L):
            idl = ids16[:, l:l + 1]
            counts = counts + (idl == cols).astype(jnp.int16)
            valid = valid + (ids[:, l:l + 1] != -1).astype(jnp.float32)
        pos_sum = jnp.dot(counts.astype(jnp.bfloat16), pt_ref[...],
                          preferred_element_type=jnp.float32)
        pooled = pos_sum / jnp.maximum(valid, EPS_)
        ttf = ids[:, L:L + 1].astype(jnp.float32)
        tt0 = tt_ref[0:1, :]
        tt1 = tt_ref[1:2, :]
        emb = ent_ref[...] + pooled + tt0 + ttf * (tt1 - tt0)
        mean = jnp.mean(emb, axis=1, keepdims=True)
        cent = emb - mean
        var = jnp.mean(cent * cent, axis=1, keepdims=True)
        o_ref[...] = (cent * lax.rsqrt(var + LN_EPS_) * g_ref[...]
                      + b_ref[...])

    return pl.pallas_call(
        body,
        grid=(nb,),
        in_specs=[
            pl.BlockSpec((R, 32), lambda i: (i, 0)),
            pl.BlockSpec((R, H), lambda i: (i, 0)),
            pl.BlockSpec((P, H), lambda i: (0, 0)),
            pl.BlockSpec((tt_table.shape[0], H), lambda i: (0, 0)),
            pl.BlockSpec((1, H), lambda i: (0, 0)),
            pl.BlockSpec((1, H), lambda i: (0, 0)),
        ],
        out_specs=pl.BlockSpec((R, H), lambda i: (i, 0)),
        out_shape=jax.ShapeDtypeStruct((BM, H), jnp.float32),
    )(ids_all, ent_emb, pos_table, tt_table, gamma, beta)


def kernel(entity_ids, position_ids, token_type_ids, entity_table,
           position_table, token_type_table, gamma, beta):
    B, M = entity_ids.shape
    L = position_ids.shape[-1]
    H = entity_table.shape[1]
    BM = B * M

    NW, CH = 32, 128
    eids = entity_ids.T.reshape(NW, BM // (NW * CH), CH).astype(jnp.int32)
    ent_emb = _entity_gather(entity_table, eids)

    pos = position_ids.transpose(1, 0, 2).reshape(BM, L).astype(jnp.int32)
    tt = token_type_ids.T.reshape(BM, 1).astype(jnp.int32)
    pad = jnp.full((BM, 32 - L - 1), -1, jnp.int32)
    ids_all = jnp.concatenate([pos, tt, pad], axis=1)

    out = _fuse(ent_emb, ids_all, position_table.astype(jnp.bfloat16),
                token_type_table, gamma.reshape(1, H), beta.reshape(1, H), L)
    return out.reshape(M, B, H).transpose(1, 0, 2)
